# Initial kernel scaffold; baseline (speedup 1.0000x reference)
import jax
import jax.numpy as jnp
from jax.experimental import pallas as pl


def kernel(x, adj_indices, adj_values, W, b, a):
    raise NotImplementedError("write your pallas kernel here")


# speedup vs baseline: 5.2511x; 5.2511x over previous
import functools

import jax
import jax.numpy as jnp
from jax import lax
from jax.experimental import pallas as pl
from jax.experimental.pallas import tpu as pltpu
from jax.experimental.pallas import tpu_sc as plsc

N = 10000
E = 320000
D = 128

NC = 2
NS = 16
NW = NC * NS
EPW = E // NW
K = 80
NCHUNK = EPW // K
BAND = 624
TAIL = N - NS * BAND


def _mm_body(x_ref, w_ref, b_ref, h_ref):
    h_ref[...] = lax.dot_general(
        x_ref[...], w_ref[...],
        (((1,), (1,)), ((), ())),
        preferred_element_type=jnp.float32,
    ) + b_ref[...]


def _matmul(x, W, b):
    grid = (10,)
    blk = N // 10
    return pl.pallas_call(
        _mm_body,
        grid=grid,
        in_specs=[
            pl.BlockSpec((blk, D), lambda i: (i, 0)),
            pl.BlockSpec((D, D), lambda i: (0, 0)),
            pl.BlockSpec((1, D), lambda i: (0, 0)),
        ],
        out_specs=pl.BlockSpec((blk, D), lambda i: (i, 0)),
        out_shape=jax.ShapeDtypeStruct((N, D), jnp.float32),
    )(x, W, b.reshape(1, D))


@functools.cache
def _make_sc_scatter():
    mesh = plsc.VectorSubcoreMesh(
        core_axis_name="c", subcore_axis_name="s",
        num_cores=NC, num_subcores=NS)
    return pl.kernel(
        _sc_scatter_body,
        mesh=mesh,
        out_type=jax.ShapeDtypeStruct((NC, N, D), jnp.float32),
        scratch_types=[
            pltpu.VMEM((NCHUNK, 1, K), jnp.int32),
            pltpu.VMEM((EPW,), jnp.int32),
            pltpu.VMEM((EPW,), jnp.float32),
            pltpu.VMEM((K, D), jnp.float32),
            pltpu.VMEM_SHARED((N, D), jnp.float32),
            pltpu.SemaphoreType.DMA,
        ],
        compiler_params=pltpu.CompilerParams(needs_layout_passes=False),
    )


def _sc_scatter_body(h_hbm, rowi_hbm, coli_hbm, val_hbm, zeros_hbm, out_hbm,
                     rowv, colv, valv, rows, acc, gsem):
    c = lax.axis_index("c")
    s = lax.axis_index("s")
    wid = s * NC + c

    pltpu.sync_copy(rowi_hbm.at[wid], rowv)
    pltpu.sync_copy(coli_hbm.at[wid], colv)
    pltpu.sync_copy(val_hbm.at[wid], valv)

    r0 = s * BAND
    pltpu.sync_copy(zeros_hbm.at[pl.ds(r0, BAND)], acc.at[pl.ds(r0, BAND)])

    @pl.when(s == 0)
    def _zero_tail():
        pltpu.sync_copy(zeros_hbm.at[pl.ds(NS * BAND, TAIL)],
                        acc.at[pl.ds(NS * BAND, TAIL)])

    plsc.subcore_barrier()

    def chunk_body(g, _):
        pltpu.async_copy(h_hbm.at[colv.at[pl.ds(g * K, K)]], rows, gsem).wait()
        def edge_body(e, _):
            sv = plsc.load_gather(valv, [jnp.full((16,), g * K + e, jnp.int32)])
            for ci in range(D // 16):
                sl = pl.ds(16 * ci, 16)
                rows[e, sl] = rows[e, sl] * sv
            return 0
        lax.fori_loop(0, K, edge_body, 0, unroll=2)
        pltpu.sync_copy(rows, acc.at[rowv.at[g, 0]], add=True)
        return 0

    lax.fori_loop(0, NCHUNK, chunk_body, 0)

    plsc.subcore_barrier()
    pltpu.sync_copy(acc.at[pl.ds(r0, BAND)], out_hbm.at[c, pl.ds(r0, BAND)])

    @pl.when(s == 0)
    def _write_tail():
        pltpu.sync_copy(acc.at[pl.ds(NS * BAND, TAIL)],
                        out_hbm.at[c, pl.ds(NS * BAND, TAIL)])


def _comb_body(p_ref, a_ref, o_ref):
    t = p_ref[0] + p_ref[1]
    o_ref[...] = jnp.where(t >= 0, t, a_ref[0, 0] * t)


def _combine(parts, a):
    grid = (10,)
    blk = N // 10
    return pl.pallas_call(
        _comb_body,
        grid=grid,
        in_specs=[
            pl.BlockSpec((NC, blk, D), lambda i: (0, i, 0)),
            pl.BlockSpec((1, 1), lambda i: (0, 0)),
        ],
        out_specs=pl.BlockSpec((blk, D), lambda i: (i, 0)),
        out_shape=jax.ShapeDtypeStruct((N, D), jnp.float32),
    )(parts, a.reshape(1, 1))


def kernel(x, adj_indices, adj_values, W, b, a):
    h = _matmul(x, W, b)
    row = adj_indices[0].astype(jnp.int32).reshape(NW, NCHUNK, 1, K)
    col = adj_indices[1].astype(jnp.int32).reshape(NW, EPW)
    val = adj_values.astype(jnp.float32).reshape(NW, EPW)
    zeros = jnp.zeros((N, D), jnp.float32)
    parts = _make_sc_scatter()(h, row, col, val, zeros)
    return _combine(parts, a.astype(jnp.float32))


# speedup vs baseline: 8.3194x; 1.5843x over previous
import functools

import jax
import jax.numpy as jnp
from jax import lax
from jax.experimental import pallas as pl
from jax.experimental.pallas import tpu as pltpu
from jax.experimental.pallas import tpu_sc as plsc

N = 10000
E = 320000
D = 128
DH = D // 2

NC = 2
NS = 16
NW = NC * NS
KM = 128
NCHUNK = 80
EPW = KM * NCHUNK
E_PAD = NW * EPW
NSLOT = 4
BAND = 624
TAIL = N - NS * BAND


def _mm_body(x_ref, w_ref, b_ref, h_ref):
    h_ref[...] = lax.dot_general(
        x_ref[...], w_ref[...],
        (((1,), (1,)), ((), ())),
        preferred_element_type=jnp.float32,
    ) + b_ref[...]


def _matmul(x, W, b):
    grid = (10,)
    blk = N // 10
    return pl.pallas_call(
        _mm_body,
        grid=grid,
        in_specs=[
            pl.BlockSpec((blk, D), lambda i: (i, 0)),
            pl.BlockSpec((D, D), lambda i: (0, 0)),
            pl.BlockSpec((1, D), lambda i: (0, 0)),
        ],
        out_specs=pl.BlockSpec((blk, D), lambda i: (i, 0)),
        out_shape=jax.ShapeDtypeStruct((N, D), jnp.float32),
    )(x, W, b.reshape(1, D))


@functools.cache
def _make_sc_scatter():
    mesh = plsc.VectorSubcoreMesh(
        core_axis_name="c", subcore_axis_name="s",
        num_cores=NC, num_subcores=NS)
    return pl.kernel(
        _sc_scatter_body,
        mesh=mesh,
        out_type=jax.ShapeDtypeStruct((NC, 2, N, DH), jnp.float32),
        scratch_types=[
            pltpu.VMEM((NCHUNK, 1, KM), jnp.int32),
            pltpu.VMEM((EPW,), jnp.int32),
            pltpu.VMEM((EPW,), jnp.float32),
            pltpu.VMEM((NSLOT, KM, DH), jnp.float32),
            pltpu.VMEM_SHARED((N, DH), jnp.float32),
            [pltpu.SemaphoreType.DMA] * NSLOT,
            [pltpu.SemaphoreType.DMA] * NSLOT,
        ],
        compiler_params=pltpu.CompilerParams(
            needs_layout_passes=False, use_tc_tiling_on_sc=False),
    )


def _sc_scatter_body(h0_hbm, h1_hbm, rowi_hbm, coli_hbm, val_hbm, zeros_hbm,
                     out_hbm, rowv, colv, valv, rows, acc, gsems, ssems):
    c = lax.axis_index("c")
    s = lax.axis_index("s")
    wid = s * NC + c
    r0 = s * BAND

    pltpu.sync_copy(rowi_hbm.at[wid], rowv)
    pltpu.sync_copy(coli_hbm.at[wid], colv)
    pltpu.sync_copy(val_hbm.at[wid], valv)

    def zero_acc():
        pltpu.sync_copy(zeros_hbm.at[pl.ds(r0, BAND)], acc.at[pl.ds(r0, BAND)])

        @pl.when(s == 0)
        def _zero_tail():
            pltpu.sync_copy(zeros_hbm.at[pl.ds(NS * BAND, TAIL)],
                            acc.at[pl.ds(NS * BAND, TAIL)])

    def run_phase(p, h_hbm):
        zero_acc()
        plsc.subcore_barrier()

        def gather_desc(g, b):
            return pltpu.make_async_copy(
                h_hbm.at[colv.at[pl.ds(g * KM, KM)]], rows.at[b], gsems[b])

        def scatter_desc(g, b):
            return pltpu.make_async_copy(
                rows.at[b], acc.at[rowv.at[g, 0]], ssems[b])

        gather_desc(0, 0).start()

        def quad_body(i, _):
            g0 = i * NSLOT
            for b in range(NSLOT):
                g = g0 + b
                nb = (b + 1) % NSLOT

                @pl.when(g + 1 - NSLOT >= 0)
                def _drain():
                    scatter_desc(g + 1 - NSLOT, nb).wait()

                @pl.when(g + 1 < NCHUNK)
                def _prefetch():
                    gather_desc(g + 1, nb).start()

                gather_desc(g, b).wait()

                @plsc.parallel_loop(0, KM, unroll=4)
                def _scale(e):
                    sv = plsc.load_gather(
                        valv, [jnp.full((16,), g * KM + e, jnp.int32)])
                    for ci in range(DH // 16):
                        sl = pl.ds(16 * ci, 16)
                        rows[b, e, sl] = rows[b, e, sl] * sv

                scatter_desc(g, b).start(add=True)
            return 0

        lax.fori_loop(0, NCHUNK // NSLOT, quad_body, 0)
        for b in range(1, NSLOT):
            scatter_desc(NCHUNK - NSLOT + b, b).wait()

        plsc.subcore_barrier()
        pltpu.sync_copy(acc.at[pl.ds(r0, BAND)],
                        out_hbm.at[c, p, pl.ds(r0, BAND)])

        @pl.when(s == 0)
        def _write_tail():
            pltpu.sync_copy(acc.at[pl.ds(NS * BAND, TAIL)],
                            out_hbm.at[c, p, pl.ds(NS * BAND, TAIL)])

    run_phase(0, h0_hbm)
    plsc.subcore_barrier()
    run_phase(1, h1_hbm)


def _comb_body(p_ref, a_ref, o_ref):
    a = a_ref[0, 0]
    t0 = p_ref[0, 0] + p_ref[1, 0]
    t1 = p_ref[0, 1] + p_ref[1, 1]
    o_ref[:, :DH] = jnp.where(t0 >= 0, t0, a * t0)
    o_ref[:, DH:] = jnp.where(t1 >= 0, t1, a * t1)


def _combine(parts, a):
    grid = (10,)
    blk = N // 10
    return pl.pallas_call(
        _comb_body,
        grid=grid,
        in_specs=[
            pl.BlockSpec((NC, 2, blk, DH), lambda i: (0, 0, i, 0)),
            pl.BlockSpec((1, 1), lambda i: (0, 0)),
        ],
        out_specs=pl.BlockSpec((blk, D), lambda i: (i, 0)),
        out_shape=jax.ShapeDtypeStruct((N, D), jnp.float32),
    )(parts, a.reshape(1, 1))


def kernel(x, adj_indices, adj_values, W, b, a):
    h = _matmul(x, W, b)
    h0 = h[:, :DH]
    h1 = h[:, DH:]
    pad = E_PAD - E
    pidx = jnp.arange(pad, dtype=jnp.int32) % N
    row = jnp.concatenate([adj_indices[0].astype(jnp.int32), pidx])
    col = jnp.concatenate([adj_indices[1].astype(jnp.int32), pidx])
    val = jnp.concatenate(
        [adj_values.astype(jnp.float32), jnp.zeros((pad,), jnp.float32)])
    zeros = jnp.zeros((N, DH), jnp.float32)
    parts = _make_sc_scatter()(
        h0, h1,
        row.reshape(NW, NCHUNK, 1, KM),
        col.reshape(NW, EPW),
        val.reshape(NW, EPW),
        zeros,
    )
    return _combine(parts, a.astype(jnp.float32))


# speedup vs baseline: 8.3785x; 1.0071x over previous
import functools

import jax
import jax.numpy as jnp
from jax import lax
from jax.experimental import pallas as pl
from jax.experimental.pallas import tpu as pltpu
from jax.experimental.pallas import tpu_sc as plsc

N = 10000
E = 320000
D = 128
DH = D // 2

NC = 2
NS = 16
NW = NC * NS
KM = 128
NCHUNK = 80
EPW = KM * NCHUNK
E_PAD = NW * EPW
NSLOT = 4
BAND = 624
TAIL = N - NS * BAND


def _mm_body(x_ref, w_ref, b_ref, h_ref):
    h_ref[...] = lax.dot_general(
        x_ref[...], w_ref[...],
        (((1,), (1,)), ((), ())),
        preferred_element_type=jnp.float32,
    ) + b_ref[...]


def _matmul(x, W, b):
    grid = (10,)
    blk = N // 10
    return pl.pallas_call(
        _mm_body,
        grid=grid,
        in_specs=[
            pl.BlockSpec((blk, D), lambda i: (i, 0)),
            pl.BlockSpec((D, D), lambda i: (0, 0)),
            pl.BlockSpec((1, D), lambda i: (0, 0)),
        ],
        out_specs=pl.BlockSpec((blk, D), lambda i: (i, 0)),
        out_shape=jax.ShapeDtypeStruct((N, D), jnp.float32),
    )(x, W, b.reshape(1, D))


@functools.cache
def _make_sc_scatter():
    mesh = plsc.VectorSubcoreMesh(
        core_axis_name="c", subcore_axis_name="s",
        num_cores=NC, num_subcores=NS)
    return pl.kernel(
        _sc_scatter_body,
        mesh=mesh,
        out_type=jax.ShapeDtypeStruct((NC, 2, N, DH), jnp.float32),
        scratch_types=[
            pltpu.VMEM((NCHUNK, 1, KM), jnp.int32),
            pltpu.VMEM((EPW,), jnp.int32),
            pltpu.VMEM((EPW,), jnp.float32),
            pltpu.VMEM((NSLOT, KM, DH), jnp.float32),
            pltpu.VMEM_SHARED((N, DH), jnp.float32),
            [pltpu.SemaphoreType.DMA] * NSLOT,
            [pltpu.SemaphoreType.DMA] * NSLOT,
        ],
        compiler_params=pltpu.CompilerParams(
            needs_layout_passes=False, use_tc_tiling_on_sc=False),
    )


def _sc_scatter_body(h0_hbm, h1_hbm, rowi_hbm, coli_hbm, val_hbm, zeros_hbm,
                     out_hbm, rowv, colv, valv, rows, acc, gsems, ssems):
    c = lax.axis_index("c")
    s = lax.axis_index("s")
    wid = s * NC + c
    r0 = s * BAND

    pltpu.sync_copy(rowi_hbm.at[wid], rowv)
    pltpu.sync_copy(coli_hbm.at[wid], colv)
    pltpu.sync_copy(val_hbm.at[wid], valv)

    def zero_acc():
        pltpu.sync_copy(zeros_hbm.at[pl.ds(r0, BAND)], acc.at[pl.ds(r0, BAND)])

        @pl.when(s == 0)
        def _zero_tail():
            pltpu.sync_copy(zeros_hbm.at[pl.ds(NS * BAND, TAIL)],
                            acc.at[pl.ds(NS * BAND, TAIL)])

    def run_phase(p, h_hbm):
        zero_acc()
        plsc.subcore_barrier()

        def gather_desc(g, b):
            return pltpu.make_async_copy(
                h_hbm.at[colv.at[pl.ds(g * KM, KM)]], rows.at[b], gsems[b])

        def scatter_desc(g, b):
            return pltpu.make_async_copy(
                rows.at[b], acc.at[rowv.at[g, 0]], ssems[b])

        gather_desc(0, 0).start()

        def quad_body(i, _):
            g0 = i * NSLOT
            for b in range(NSLOT):
                g = g0 + b
                nb = (b + 1) % NSLOT

                @pl.when(g + 1 - NSLOT >= 0)
                def _drain():
                    scatter_desc(g + 1 - NSLOT, nb).wait()

                @pl.when(g + 1 < NCHUNK)
                def _prefetch():
                    gather_desc(g + 1, nb).start()

                gather_desc(g, b).wait()

                @plsc.parallel_loop(0, KM, step=16, unroll=2)
                def _scale(e0):
                    vals16 = valv[pl.ds(g * KM + e0, 16)]
                    for j in range(16):
                        sv = jnp.broadcast_to(vals16[j], (16,))
                        for ci in range(DH // 16):
                            sl = pl.ds(16 * ci, 16)
                            rows[b, e0 + j, sl] = rows[b, e0 + j, sl] * sv

                scatter_desc(g, b).start(add=True)
            return 0

        lax.fori_loop(0, NCHUNK // NSLOT, quad_body, 0)
        for b in range(1, NSLOT):
            scatter_desc(NCHUNK - NSLOT + b, b).wait()

        plsc.subcore_barrier()
        pltpu.sync_copy(acc.at[pl.ds(r0, BAND)],
                        out_hbm.at[c, p, pl.ds(r0, BAND)])

        @pl.when(s == 0)
        def _write_tail():
            pltpu.sync_copy(acc.at[pl.ds(NS * BAND, TAIL)],
                            out_hbm.at[c, p, pl.ds(NS * BAND, TAIL)])

    run_phase(0, h0_hbm)
    plsc.subcore_barrier()
    run_phase(1, h1_hbm)


def _comb_body(p_ref, a_ref, o_ref):
    a = a_ref[0, 0]
    t0 = p_ref[0, 0] + p_ref[1, 0]
    t1 = p_ref[0, 1] + p_ref[1, 1]
    o_ref[:, :DH] = jnp.where(t0 >= 0, t0, a * t0)
    o_ref[:, DH:] = jnp.where(t1 >= 0, t1, a * t1)


def _combine(parts, a):
    grid = (10,)
    blk = N // 10
    return pl.pallas_call(
        _comb_body,
        grid=grid,
        in_specs=[
            pl.BlockSpec((NC, 2, blk, DH), lambda i: (0, 0, i, 0)),
            pl.BlockSpec((1, 1), lambda i: (0, 0)),
        ],
        out_specs=pl.BlockSpec((blk, D), lambda i: (i, 0)),
        out_shape=jax.ShapeDtypeStruct((N, D), jnp.float32),
    )(parts, a.reshape(1, 1))


def kernel(x, adj_indices, adj_values, W, b, a):
    h = _matmul(x, W, b)
    h0 = h[:, :DH]
    h1 = h[:, DH:]
    pad = E_PAD - E
    pidx = jnp.arange(pad, dtype=jnp.int32) % N
    row = jnp.concatenate([adj_indices[0].astype(jnp.int32), pidx])
    col = jnp.concatenate([adj_indices[1].astype(jnp.int32), pidx])
    val = jnp.concatenate(
        [adj_values.astype(jnp.float32), jnp.zeros((pad,), jnp.float32)])
    zeros = jnp.zeros((N, DH), jnp.float32)
    parts = _make_sc_scatter()(
        h0, h1,
        row.reshape(NW, NCHUNK, 1, KM),
        col.reshape(NW, EPW),
        val.reshape(NW, EPW),
        zeros,
    )
    return _combine(parts, a.astype(jnp.float32))


# speedup vs baseline: 8.7223x; 1.0410x over previous
import functools

import jax
import jax.numpy as jnp
from jax import lax
from jax.experimental import pallas as pl
from jax.experimental.pallas import tpu as pltpu
from jax.experimental.pallas import tpu_sc as plsc

N = 10000
E = 320000
D = 128
DH = D // 2

NC = 2
NS = 16
NW = NC * NS
KM = 128
NCHUNK = 80
EPW = KM * NCHUNK
NSLOT = 4
BAND = 624
TAIL = N - NS * BAND
EREAL = E - (NW - 1) * EPW
EFILL = EPW - EREAL


def _mm_body(x_ref, w_ref, b_ref, h0_ref, h1_ref):
    h = lax.dot_general(
        x_ref[...], w_ref[...],
        (((1,), (1,)), ((), ())),
        preferred_element_type=jnp.float32,
    ) + b_ref[...]
    h0_ref[...] = h[:, :DH]
    h1_ref[...] = h[:, DH:]


def _matmul(x, W, b):
    grid = (10,)
    blk = N // 10
    return pl.pallas_call(
        _mm_body,
        grid=grid,
        in_specs=[
            pl.BlockSpec((blk, D), lambda i: (i, 0)),
            pl.BlockSpec((D, D), lambda i: (0, 0)),
            pl.BlockSpec((1, D), lambda i: (0, 0)),
        ],
        out_specs=[
            pl.BlockSpec((blk, DH), lambda i: (i, 0)),
            pl.BlockSpec((blk, DH), lambda i: (i, 0)),
        ],
        out_shape=[
            jax.ShapeDtypeStruct((N, DH), jnp.float32),
            jax.ShapeDtypeStruct((N, DH), jnp.float32),
        ],
    )(x, W, b.reshape(1, D))


@functools.cache
def _make_sc_scatter():
    mesh = plsc.VectorSubcoreMesh(
        core_axis_name="c", subcore_axis_name="s",
        num_cores=NC, num_subcores=NS)
    return pl.kernel(
        _sc_scatter_body,
        mesh=mesh,
        out_type=jax.ShapeDtypeStruct((NC, 2, N, DH), jnp.float32),
        scratch_types=[
            pltpu.VMEM((EPW,), jnp.int32),
            pltpu.VMEM((EPW,), jnp.int32),
            pltpu.VMEM((EPW,), jnp.float32),
            pltpu.VMEM((NSLOT, KM, DH), jnp.float32),
            pltpu.VMEM_SHARED((N, DH), jnp.float32),
            [pltpu.SemaphoreType.DMA] * NSLOT,
            [pltpu.SemaphoreType.DMA] * NSLOT,
        ],
        compiler_params=pltpu.CompilerParams(
            needs_layout_passes=False, use_tc_tiling_on_sc=False),
    )


def _sc_scatter_body(h0_hbm, h1_hbm, rowi_hbm, coli_hbm, val_hbm, zeros_hbm,
                     out_hbm, rowv, colv, valv, rows, acc, gsems, ssems):
    c = lax.axis_index("c")
    s = lax.axis_index("s")
    wid = s * NC + c
    r0 = s * BAND

    @pl.when(wid < NW - 1)
    def _stage_full():
        base = wid * EPW
        pltpu.sync_copy(rowi_hbm.at[pl.ds(base, EPW)], rowv)
        pltpu.sync_copy(coli_hbm.at[pl.ds(base, EPW)], colv)
        pltpu.sync_copy(val_hbm.at[pl.ds(base, EPW)], valv)

    @pl.when(wid == NW - 1)
    def _stage_tail():
        base = (NW - 1) * EPW
        pltpu.sync_copy(rowi_hbm.at[pl.ds(base, EREAL)],
                        rowv.at[pl.ds(0, EREAL)])
        pltpu.sync_copy(coli_hbm.at[pl.ds(base, EREAL)],
                        colv.at[pl.ds(0, EREAL)])
        pltpu.sync_copy(val_hbm.at[pl.ds(base, EREAL)],
                        valv.at[pl.ds(0, EREAL)])
        zero16 = jnp.zeros((16,), jnp.float32)
        iota16 = lax.iota(jnp.int32, 16)

        @plsc.parallel_loop(0, EFILL, step=16)
        def _fill(i):
            spread = iota16 + i
            rowv[pl.ds(EREAL + i, 16)] = spread
            colv[pl.ds(EREAL + i, 16)] = spread
            valv[pl.ds(EREAL + i, 16)] = zero16

    def zero_acc():
        pltpu.sync_copy(zeros_hbm.at[pl.ds(r0, BAND)], acc.at[pl.ds(r0, BAND)])

        @pl.when(s == 0)
        def _zero_tail():
            pltpu.sync_copy(zeros_hbm.at[pl.ds(NS * BAND, TAIL)],
                            acc.at[pl.ds(NS * BAND, TAIL)])

    def run_phase(p, h_hbm):
        zero_acc()
        plsc.subcore_barrier()

        def gather_desc(g, b):
            return pltpu.make_async_copy(
                h_hbm.at[colv.at[pl.ds(g * KM, KM)]], rows.at[b], gsems[b])

        def scatter_desc(g, b):
            return pltpu.make_async_copy(
                rows.at[b], acc.at[rowv.at[pl.ds(g * KM, KM)]], ssems[b])

        gather_desc(0, 0).start()

        def quad_body(i, _):
            g0 = i * NSLOT
            for b in range(NSLOT):
                g = g0 + b
                nb = (b + 1) % NSLOT

                @pl.when(g + 1 - NSLOT >= 0)
                def _drain():
                    scatter_desc(g + 1 - NSLOT, nb).wait()

                @pl.when(g + 1 < NCHUNK)
                def _prefetch():
                    gather_desc(g + 1, nb).start()

                gather_desc(g, b).wait()

                @plsc.parallel_loop(0, KM, step=16, unroll=2)
                def _scale(e0):
                    vals16 = valv[pl.ds(g * KM + e0, 16)]
                    for j in range(16):
                        sv = jnp.broadcast_to(vals16[j], (16,))
                        for ci in range(DH // 16):
                            sl = pl.ds(16 * ci, 16)
                            rows[b, e0 + j, sl] = rows[b, e0 + j, sl] * sv

                scatter_desc(g, b).start(add=True)
            return 0

        lax.fori_loop(0, NCHUNK // NSLOT, quad_body, 0)
        for b in range(1, NSLOT):
            scatter_desc(NCHUNK - NSLOT + b, b).wait()

        plsc.subcore_barrier()
        pltpu.sync_copy(acc.at[pl.ds(r0, BAND)],
                        out_hbm.at[c, p, pl.ds(r0, BAND)])

        @pl.when(s == 0)
        def _write_tail():
            pltpu.sync_copy(acc.at[pl.ds(NS * BAND, TAIL)],
                            out_hbm.at[c, p, pl.ds(NS * BAND, TAIL)])

    run_phase(0, h0_hbm)
    plsc.subcore_barrier()
    run_phase(1, h1_hbm)


def _comb_body(p_ref, a_ref, o_ref):
    a = a_ref[0, 0]
    t0 = p_ref[0, 0] + p_ref[1, 0]
    t1 = p_ref[0, 1] + p_ref[1, 1]
    o_ref[:, :DH] = jnp.where(t0 >= 0, t0, a * t0)
    o_ref[:, DH:] = jnp.where(t1 >= 0, t1, a * t1)


def _combine(parts, a):
    grid = (10,)
    blk = N // 10
    return pl.pallas_call(
        _comb_body,
        grid=grid,
        in_specs=[
            pl.BlockSpec((NC, 2, blk, DH), lambda i: (0, 0, i, 0)),
            pl.BlockSpec((1, 1), lambda i: (0, 0)),
        ],
        out_specs=pl.BlockSpec((blk, D), lambda i: (i, 0)),
        out_shape=jax.ShapeDtypeStruct((N, D), jnp.float32),
    )(parts, a.reshape(1, 1))


def kernel(x, adj_indices, adj_values, W, b, a):
    h0, h1 = _matmul(x, W, b)
    zeros = jnp.zeros((N, DH), jnp.float32)
    parts = _make_sc_scatter()(
        h0, h1,
        adj_indices[0].astype(jnp.int32),
        adj_indices[1].astype(jnp.int32),
        adj_values.astype(jnp.float32),
        zeros,
    )
    return _combine(parts, a.astype(jnp.float32))


# speedup vs baseline: 11.1978x; 1.2838x over previous
import functools

import jax
import jax.numpy as jnp
from jax import lax
from jax.experimental import pallas as pl
from jax.experimental.pallas import tpu as pltpu
from jax.experimental.pallas import tpu_sc as plsc

N = 10000
E = 320000
D = 128
DH = D // 2

NC = 2
NS = 16
KM = 128
NCHUNK = 159
EPW = KM * NCHUNK
NSLOT = 3
BAND = 624
TAIL = N - NS * BAND
EREAL = E - (NS - 1) * EPW
EFILL = EPW - EREAL
WCH = 128


def _mm_body(x_ref, w_ref, b_ref, h_ref):
    h_ref[...] = lax.dot_general(
        x_ref[...], w_ref[...],
        (((1,), (1,)), ((), ())),
        preferred_element_type=jnp.float32,
    ) + b_ref[...]


def _matmul(x, W, b):
    grid = (10,)
    blk = N // 10
    return pl.pallas_call(
        _mm_body,
        grid=grid,
        in_specs=[
            pl.BlockSpec((blk, D), lambda i: (i, 0)),
            pl.BlockSpec((D, D), lambda i: (0, 0)),
            pl.BlockSpec((1, D), lambda i: (0, 0)),
        ],
        out_specs=pl.BlockSpec((blk, D), lambda i: (i, 0)),
        out_shape=jax.ShapeDtypeStruct((N, D), jnp.float32),
    )(x, W, b.reshape(1, D))


@functools.cache
def _make_sc_scatter():
    mesh = plsc.VectorSubcoreMesh(
        core_axis_name="c", subcore_axis_name="s",
        num_cores=NC, num_subcores=NS)
    return pl.kernel(
        _sc_scatter_body,
        mesh=mesh,
        out_type=jax.ShapeDtypeStruct((N, D), jnp.float32),
        scratch_types=[
            pltpu.VMEM((EPW,), jnp.int32),
            pltpu.VMEM((EPW,), jnp.int32),
            pltpu.VMEM((EPW,), jnp.float32),
            pltpu.VMEM((NSLOT, KM, DH), jnp.float32),
            pltpu.VMEM((16,), jnp.float32),
            pltpu.VMEM_SHARED((N, DH), jnp.float32),
            [pltpu.SemaphoreType.DMA] * NSLOT,
            [pltpu.SemaphoreType.DMA] * NSLOT,
        ],
        compiler_params=pltpu.CompilerParams(
            needs_layout_passes=False, use_tc_tiling_on_sc=False),
    )


def _sc_scatter_body(h2_hbm, adj_hbm, val_hbm, zeros_hbm, a_hbm,
                     out_hbm, rowv, colv, valv, rows, av, acc,
                     gsems, ssems):
    c = lax.axis_index("c")
    s = lax.axis_index("s")
    r0 = s * BAND

    @pl.when(s < NS - 1)
    def _stage_full():
        base = s * EPW
        pltpu.sync_copy(adj_hbm.at[0, pl.ds(base, EPW)], rowv)
        pltpu.sync_copy(adj_hbm.at[1, pl.ds(base, EPW)], colv)
        pltpu.sync_copy(val_hbm.at[pl.ds(base, EPW)], valv)

    @pl.when(s == NS - 1)
    def _stage_tail():
        base = (NS - 1) * EPW
        pltpu.sync_copy(adj_hbm.at[0, pl.ds(base, EREAL)],
                        rowv.at[pl.ds(0, EREAL)])
        pltpu.sync_copy(adj_hbm.at[1, pl.ds(base, EREAL)],
                        colv.at[pl.ds(0, EREAL)])
        pltpu.sync_copy(val_hbm.at[pl.ds(base, EREAL)],
                        valv.at[pl.ds(0, EREAL)])
        zero16 = jnp.zeros((16,), jnp.float32)
        iota16 = lax.iota(jnp.int32, 16)

        @plsc.parallel_loop(0, EFILL, step=16)
        def _fill(i):
            spread = iota16 + i
            rowv[pl.ds(EREAL + i, 16)] = spread
            colv[pl.ds(EREAL + i, 16)] = spread
            valv[pl.ds(EREAL + i, 16)] = zero16

    pltpu.sync_copy(a_hbm, av)

    @plsc.parallel_loop(0, EPW, step=16)
    def _xform(i):
        colv[pl.ds(i, 16)] = colv[pl.ds(i, 16)] * 2 + c

    pltpu.sync_copy(zeros_hbm.at[pl.ds(r0, BAND)], acc.at[pl.ds(r0, BAND)])

    @pl.when(s == 0)
    def _zero_tail():
        pltpu.sync_copy(zeros_hbm.at[pl.ds(NS * BAND, TAIL)],
                        acc.at[pl.ds(NS * BAND, TAIL)])

    plsc.subcore_barrier()

    def gather_desc(g, b):
        return pltpu.make_async_copy(
            h2_hbm.at[colv.at[pl.ds(g * KM, KM)]], rows.at[b], gsems[b])

    def scatter_desc(g, b):
        return pltpu.make_async_copy(
            rows.at[b], acc.at[rowv.at[pl.ds(g * KM, KM)]], ssems[b])

    gather_desc(0, 0).start()

    def tri_body(i, _):
        g0 = i * NSLOT
        for b in range(NSLOT):
            g = g0 + b
            nb = (b + 1) % NSLOT

            @pl.when(g + 1 - NSLOT >= 0)
            def _drain():
                scatter_desc(g + 1 - NSLOT, nb).wait()

            @pl.when(g + 1 < NCHUNK)
            def _prefetch():
                gather_desc(g + 1, nb).start()

            gather_desc(g, b).wait()

            @plsc.parallel_loop(0, KM, step=16, unroll=2)
            def _scale(e0):
                vals16 = valv[pl.ds(g * KM + e0, 16)]
                for j in range(16):
                    sv = jnp.broadcast_to(vals16[j], (16,))
                    for ci in range(DH // 16):
                        sl = pl.ds(16 * ci, 16)
                        rows[b, e0 + j, sl] = rows[b, e0 + j, sl] * sv

            scatter_desc(g, b).start(add=True)
        return 0

    lax.fori_loop(0, NCHUNK // NSLOT, tri_body, 0)
    for b in range(1, NSLOT):
        scatter_desc(NCHUNK - NSLOT + b, b).wait()

    plsc.subcore_barrier()

    alpha = av[...]
    co = c * DH
    wbuf = rows.at[0]

    def write_rows(wr0, nrows):
        pltpu.sync_copy(acc.at[pl.ds(wr0, nrows)], wbuf.at[pl.ds(0, nrows)])

        @plsc.parallel_loop(0, nrows, unroll=2)
        def _prelu(r):
            for ci in range(DH // 16):
                sl = pl.ds(16 * ci, 16)
                t = wbuf[r, sl]
                wbuf[r, sl] = jnp.where(t >= 0, t, alpha * t)

        pltpu.sync_copy(wbuf.at[pl.ds(0, nrows)],
                        out_hbm.at[pl.ds(wr0, nrows), pl.ds(co, DH)])

    for w in range(BAND // WCH):
        write_rows(r0 + w * WCH, WCH)
    write_rows(r0 + (BAND // WCH) * WCH, BAND % WCH)

    @pl.when(s == 0)
    def _write_tail():
        write_rows(NS * BAND, TAIL)


def kernel(x, adj_indices, adj_values, W, b, a):
    h = _matmul(x, W, b)
    h2 = h.reshape(2 * N, DH)
    zeros = jnp.zeros((N, DH), jnp.float32)
    a16 = jnp.full((16,), a, jnp.float32)
    return _make_sc_scatter()(
        h2,
        adj_indices.astype(jnp.int32),
        adj_values.astype(jnp.float32),
        zeros,
        a16,
    )
